# baseline (speedup 1.0000x reference)
import functools

import jax
import jax.numpy as jnp
from jax.experimental import pallas as pl
from jax.experimental.pallas import tpu as pltpu

B, V, E = 1024, 100000, 16

BB = 256
VC = 512
NV = (V + VC - 1) // VC


def _mm_body(x_ref, e_ref, o_ref):
    k = pl.program_id(1)

    @pl.when(k == 0)
    def _():
        o_ref[...] = jnp.zeros_like(o_ref)

    @pl.when(k < NV - 1)
    def _():
        o_ref[...] += jnp.dot(x_ref[...], e_ref[...],
                              preferred_element_type=jnp.float32)

    @pl.when(k == NV - 1)
    def _():
        row = jax.lax.broadcasted_iota(jnp.int32, (VC, E), 0)
        em = jnp.where(k * VC + row < V, e_ref[...], 0.0)
        o_ref[...] += jnp.dot(x_ref[...], em,
                              preferred_element_type=jnp.float32)


@jax.jit
def kernel(x, embedding):
    return pl.pallas_call(
        _mm_body,
        grid=(B // BB, NV),
        in_specs=[
            pl.BlockSpec((BB, VC), lambda b, k: (b, k)),
            pl.BlockSpec((VC, E), lambda b, k: (k, 0)),
        ],
        out_specs=pl.BlockSpec((BB, E), lambda b, k: (b, 0)),
        out_shape=jax.ShapeDtypeStruct((B, E), jnp.float32),
        compiler_params=pltpu.CompilerParams(
            dimension_semantics=("parallel", "arbitrary"),
        ),
    )(x, embedding)


# speedup vs baseline: 1.6962x; 1.6962x over previous
import functools

import jax
import jax.numpy as jnp
from jax.experimental import pallas as pl
from jax.experimental.pallas import tpu as pltpu

B, V, E = 1024, 100000, 16

BB = 512
VC = 2048
NV = (V + VC - 1) // VC


def _mm_body(x_ref, e_ref, o_ref):
    k = pl.program_id(1)

    @pl.when(k == 0)
    def _():
        o_ref[...] = jnp.zeros_like(o_ref)

    @pl.when(k < NV - 1)
    def _():
        o_ref[...] += jnp.dot(x_ref[...], e_ref[...],
                              preferred_element_type=jnp.float32)

    @pl.when(k == NV - 1)
    def _():
        row = jax.lax.broadcasted_iota(jnp.int32, (VC, E), 0)
        em = jnp.where(k * VC + row < V, e_ref[...], 0.0)
        o_ref[...] += jnp.dot(x_ref[...], em,
                              preferred_element_type=jnp.float32)


@jax.jit
def kernel(x, embedding):
    return pl.pallas_call(
        _mm_body,
        grid=(B // BB, NV),
        in_specs=[
            pl.BlockSpec((BB, VC), lambda b, k: (b, k)),
            pl.BlockSpec((VC, E), lambda b, k: (k, 0)),
        ],
        out_specs=pl.BlockSpec((BB, E), lambda b, k: (b, 0)),
        out_shape=jax.ShapeDtypeStruct((B, E), jnp.float32),
        compiler_params=pltpu.CompilerParams(
            dimension_semantics=("parallel", "arbitrary"),
        ),
    )(x, embedding)


# speedup vs baseline: 7.6350x; 4.5011x over previous
import jax
import jax.numpy as jnp
from jax.experimental import pallas as pl
from jax.experimental.pallas import tpu as pltpu

B, V, E = 1024, 100000, 16

VC = 2048
NV = (V + VC - 1) // VC


def _mm_body(e_ref, x_ref, o_ref):
    k = pl.program_id(0)

    @pl.when(k == 0)
    def _():
        o_ref[...] = jnp.zeros_like(o_ref)

    col = jax.lax.broadcasted_iota(jnp.int32, (E, VC), 1)
    em = jnp.where(k * VC + col < V, e_ref[...], 0.0)
    o_ref[...] += jnp.dot(em, x_ref[...], preferred_element_type=jnp.float32)


@jax.jit
def kernel(x, embedding):
    out_t = pl.pallas_call(
        _mm_body,
        grid=(NV,),
        in_specs=[
            pl.BlockSpec((E, VC), lambda k: (0, k)),
            pl.BlockSpec((VC, B), lambda k: (k, 0)),
        ],
        out_specs=pl.BlockSpec((E, B), lambda k: (0, 0)),
        out_shape=jax.ShapeDtypeStruct((E, B), jnp.float32),
        compiler_params=pltpu.CompilerParams(
            dimension_semantics=("arbitrary",),
        ),
    )(embedding.T, x.T)
    return out_t.T
